# baseline (speedup 1.0000x reference)
import jax
import jax.numpy as jnp
from jax.experimental import pallas as pl
from jax.experimental.pallas import tpu as pltpu

_NLAYER = 3
_L = 24
_LPAD = 32
_NTILE = 8


def _gnn_body(xn_hbm, adj_hbm, ws_ref, wn_ref, b_ref, u_hbm, v_hbm,
              xn_ref, adj_ref, as_ref, u_s, v_s, in_sems, adj_sem, out_sems):
    pltpu.make_async_copy(adj_hbm, adj_ref, adj_sem).start()
    for t in range(_NTILE):
        sl = pl.ds(t * 128, 128)
        pltpu.make_async_copy(xn_hbm.at[:, sl], xn_ref.at[:, sl],
                              in_sems.at[t]).start()

    def bd(W):
        Wp = jnp.pad(W, ((0, 0), (0, _LPAD - _L), (0, _LPAD - _L)))
        z = jnp.zeros_like(Wp)
        rows = [jnp.concatenate([Wp if c == r else z for c in range(4)], axis=2)
                for r in range(4)]
        return jnp.concatenate(rows, axis=1)

    WsB = bd(ws_ref[...])
    WnB = bd(wn_ref[...])
    bB = jnp.tile(jnp.pad(b_ref[...], ((0, 0), (0, _LPAD - _L))), (1, 4))

    pltpu.make_async_copy(adj_hbm, adj_ref, adj_sem).wait()
    A = adj_ref[...]
    Ab = (A != 0).astype(jnp.float32)
    deg = jnp.sum(Ab, axis=0)
    deg_inv = jnp.where(deg > 0, 1.0 / jnp.maximum(deg, 1.0), 0.0)
    as_ref[...] = Ab * deg_inv[None, :]

    dnT = (((0,), (0,)), ((), ()))
    dnW = (((1,), (1,)), ((), ()))

    A_s = as_ref[...]
    for t in range(_NTILE):
        sl = pl.ds(t * 128, 128)
        pltpu.make_async_copy(xn_hbm.at[:, sl], xn_ref.at[:, sl],
                              in_sems.at[t]).wait()
        Xt = xn_ref[:, sl]
        U = 4.0 * Xt
        V = jax.lax.dot_general(A, Xt, dnT,
                                preferred_element_type=jnp.float32)
        for i in range(_NLAYER):
            AU = jax.lax.dot_general(A_s, U, dnT,
                                     preferred_element_type=jnp.float32)
            AV = jax.lax.dot_general(A_s, V, dnT,
                                     preferred_element_type=jnp.float32)
            U = (jax.lax.dot_general(U, WsB[i], dnW,
                                     preferred_element_type=jnp.float32)
                 + jax.lax.dot_general(AU, WnB[i], dnW,
                                       preferred_element_type=jnp.float32)
                 + bB[i][None, :])
            V = (jax.lax.dot_general(V, WsB[i], dnW,
                                     preferred_element_type=jnp.float32)
                 + jax.lax.dot_general(AV, WnB[i], dnW,
                                       preferred_element_type=jnp.float32)
                 + bB[i][None, :])
        u_s[:, sl] = U
        v_s[:, sl] = V
        pltpu.make_async_copy(u_s.at[:, sl], u_hbm.at[:, sl],
                              out_sems.at[2 * t]).start()
        pltpu.make_async_copy(v_s.at[:, sl], v_hbm.at[:, sl],
                              out_sems.at[2 * t + 1]).start()

    for t in range(_NTILE):
        sl = pl.ds(t * 128, 128)
        pltpu.make_async_copy(u_s.at[:, sl], u_hbm.at[:, sl],
                              out_sems.at[2 * t]).wait()
        pltpu.make_async_copy(v_s.at[:, sl], v_hbm.at[:, sl],
                              out_sems.at[2 * t + 1]).wait()


def kernel(x, adj, W_self, b_self, W_neigh):
    nS, nC, nN, L = x.shape
    nG = nC * nS

    Xn = jnp.transpose(x, (2, 0, 1, 3))
    Xn = jnp.pad(Xn, ((0, 0), (0, 0), (0, 0), (0, _LPAD - L)))
    Xn = Xn.reshape(nN, nG * _LPAD)

    U, V = pl.pallas_call(
        _gnn_body,
        in_specs=[
            pl.BlockSpec(memory_space=pl.ANY),
            pl.BlockSpec(memory_space=pl.ANY),
            pl.BlockSpec(memory_space=pltpu.VMEM),
            pl.BlockSpec(memory_space=pltpu.VMEM),
            pl.BlockSpec(memory_space=pltpu.VMEM),
        ],
        out_specs=[
            pl.BlockSpec(memory_space=pl.ANY),
            pl.BlockSpec(memory_space=pl.ANY),
        ],
        out_shape=[
            jax.ShapeDtypeStruct((nN, nG * _LPAD), jnp.float32),
            jax.ShapeDtypeStruct((nN, nG * _LPAD), jnp.float32),
        ],
        scratch_shapes=[
            pltpu.VMEM((nN, nG * _LPAD), jnp.float32),
            pltpu.VMEM((nN, nN), jnp.float32),
            pltpu.VMEM((nN, nN), jnp.float32),
            pltpu.VMEM((nN, nG * _LPAD), jnp.float32),
            pltpu.VMEM((nN, nG * _LPAD), jnp.float32),
            pltpu.SemaphoreType.DMA((_NTILE,)),
            pltpu.SemaphoreType.DMA,
            pltpu.SemaphoreType.DMA((2 * _NTILE,)),
        ],
    )(Xn, adj, W_self, W_neigh, b_self)

    Ur = U.reshape(nN, nS, nC, _LPAD)[..., :L].transpose(1, 2, 0, 3)
    Vr = V.reshape(nN, nS, nC, _LPAD)[..., :L].transpose(1, 2, 0, 3)
    out = jnp.stack([Ur, Vr], axis=2).reshape(nS, 2 * nC, nN, L)
    return out


# speedup vs baseline: 1.4306x; 1.4306x over previous
import jax
import jax.numpy as jnp
from jax.experimental import pallas as pl
from jax.experimental.pallas import tpu as pltpu

_NLAYER = 3
_L = 24
_LPAD = 32
_NTILE = 8


def _gnn_body(xn_hbm, adj_hbm, ws_ref, wn_ref, b_ref, u_hbm, v_hbm,
              xn_ref, adj_ref, u_s, v_s, xn_sem, adj_sem, out_sems):
    pltpu.make_async_copy(adj_hbm, adj_ref, adj_sem).start()
    pltpu.make_async_copy(xn_hbm, xn_ref, xn_sem).start()

    def bd(W):
        Wp = jnp.pad(W, ((0, 0), (0, _LPAD - _L), (0, _LPAD - _L)))
        z = jnp.zeros_like(Wp)
        rows = [jnp.concatenate([Wp if c == r else z for c in range(4)], axis=2)
                for r in range(4)]
        return jnp.concatenate(rows, axis=1)

    WsB = bd(ws_ref[...])
    WnB = bd(wn_ref[...])
    bB = jnp.tile(jnp.pad(b_ref[...], ((0, 0), (0, _LPAD - _L))), (1, 32))

    pltpu.make_async_copy(adj_hbm, adj_ref, adj_sem).wait()
    A = adj_ref[...]
    Ab = (A != 0).astype(jnp.float32)
    deg = jnp.sum(Ab, axis=0)
    deg_inv = jnp.where(deg > 0, 1.0 / jnp.maximum(deg, 1.0), 0.0)
    A_s = Ab * deg_inv[None, :]

    dnT = (((0,), (0,)), ((), ()))
    dnW = (((1,), (1,)), ((), ()))

    def aggT(H):
        return jax.lax.dot_general(A_s, H, dnT,
                                   preferred_element_type=jnp.float32)

    def lin(H, W2):
        cols = [
            jax.lax.dot_general(H[:, 128 * t:128 * (t + 1)], W2, dnW,
                                preferred_element_type=jnp.float32)
            for t in range(_NTILE)
        ]
        return jnp.concatenate(cols, axis=1)

    pltpu.make_async_copy(xn_hbm, xn_ref, xn_sem).wait()
    Xn = xn_ref[...]

    T = aggT(Xn)
    U = 4.0 * Xn
    V = deg[:, None] * T
    AV = aggT(V)
    U = lin(U, WsB[0]) + lin(4.0 * T, WnB[0]) + bB[0][None, :]
    V = lin(V, WsB[0]) + lin(AV, WnB[0]) + bB[0][None, :]
    for i in range(1, _NLAYER):
        AU = aggT(U)
        AV = aggT(V)
        U = lin(U, WsB[i]) + lin(AU, WnB[i]) + bB[i][None, :]
        V = lin(V, WsB[i]) + lin(AV, WnB[i]) + bB[i][None, :]

    u_s[...] = U
    pltpu.make_async_copy(u_s, u_hbm, out_sems.at[0]).start()
    v_s[...] = V
    pltpu.make_async_copy(v_s, v_hbm, out_sems.at[1]).start()
    pltpu.make_async_copy(u_s, u_hbm, out_sems.at[0]).wait()
    pltpu.make_async_copy(v_s, v_hbm, out_sems.at[1]).wait()


def kernel(x, adj, W_self, b_self, W_neigh):
    nS, nC, nN, L = x.shape
    nG = nC * nS

    Xn = jnp.transpose(x, (2, 0, 1, 3))
    Xn = jnp.pad(Xn, ((0, 0), (0, 0), (0, 0), (0, _LPAD - L)))
    Xn = Xn.reshape(nN, nG * _LPAD)

    U, V = pl.pallas_call(
        _gnn_body,
        in_specs=[
            pl.BlockSpec(memory_space=pl.ANY),
            pl.BlockSpec(memory_space=pl.ANY),
            pl.BlockSpec(memory_space=pltpu.VMEM),
            pl.BlockSpec(memory_space=pltpu.VMEM),
            pl.BlockSpec(memory_space=pltpu.VMEM),
        ],
        out_specs=[
            pl.BlockSpec(memory_space=pl.ANY),
            pl.BlockSpec(memory_space=pl.ANY),
        ],
        out_shape=[
            jax.ShapeDtypeStruct((nN, nG * _LPAD), jnp.float32),
            jax.ShapeDtypeStruct((nN, nG * _LPAD), jnp.float32),
        ],
        scratch_shapes=[
            pltpu.VMEM((nN, nG * _LPAD), jnp.float32),
            pltpu.VMEM((nN, nN), jnp.float32),
            pltpu.VMEM((nN, nG * _LPAD), jnp.float32),
            pltpu.VMEM((nN, nG * _LPAD), jnp.float32),
            pltpu.SemaphoreType.DMA,
            pltpu.SemaphoreType.DMA,
            pltpu.SemaphoreType.DMA((2,)),
        ],
    )(Xn, adj, W_self, W_neigh, b_self)

    Ur = U.reshape(nN, nS, nC, _LPAD)[..., :L].transpose(1, 2, 0, 3)
    Vr = V.reshape(nN, nS, nC, _LPAD)[..., :L].transpose(1, 2, 0, 3)
    out = jnp.stack([Ur, Vr], axis=2).reshape(nS, 2 * nC, nN, L)
    return out
